# baseline (speedup 1.0000x reference)
import functools

import jax
import jax.numpy as jnp
from jax import lax
from jax.experimental import pallas as pl
from jax.experimental.pallas import tpu as pltpu
from jax.experimental.pallas import tpu_sc as plsc

B = 16384
D = 64
H = 128
NC = 2
NS = 16
NW = NC * NS
BPW = B // NW


def _sc_gather_body(item_id, cat1, cat2, cat3,
                    item_table, c1_table, c2_table, c3_table,
                    e_out,
                    idx0, idx1, idx2, idx3, rows0, rows1, sem0, sem1):
    wid = lax.axis_index("s") * NC + lax.axis_index("c")
    base = wid * BPW

    pltpu.sync_copy(item_id.at[pl.ds(base, BPW)], idx0)
    pltpu.sync_copy(cat1.at[pl.ds(base, BPW)], idx1)
    pltpu.sync_copy(cat2.at[pl.ds(base, BPW)], idx2)
    pltpu.sync_copy(cat3.at[pl.ds(base, BPW)], idx3)

    cp0 = pltpu.async_copy(item_table.at[idx0], rows0, sem0)
    cp1 = pltpu.async_copy(c1_table.at[idx1], rows1, sem1)
    cp0.wait()
    pltpu.sync_copy(rows0, e_out.at[0, pl.ds(base, BPW)])
    cp2 = pltpu.async_copy(c2_table.at[idx2], rows0, sem0)
    cp1.wait()
    pltpu.sync_copy(rows1, e_out.at[1, pl.ds(base, BPW)])
    cp3 = pltpu.async_copy(c3_table.at[idx3], rows1, sem1)
    cp2.wait()
    pltpu.sync_copy(rows0, e_out.at[2, pl.ds(base, BPW)])
    cp3.wait()
    pltpu.sync_copy(rows1, e_out.at[3, pl.ds(base, BPW)])


@functools.cache
def _sc_gather():
    return pl.kernel(
        _sc_gather_body,
        out_type=jax.ShapeDtypeStruct((4, B, D), jnp.float32),
        mesh=plsc.VectorSubcoreMesh(core_axis_name="c", subcore_axis_name="s"),
        scratch_types=[
            pltpu.VMEM((BPW,), jnp.int32),
            pltpu.VMEM((BPW,), jnp.int32),
            pltpu.VMEM((BPW,), jnp.int32),
            pltpu.VMEM((BPW,), jnp.int32),
            pltpu.VMEM((BPW, D), jnp.float32),
            pltpu.VMEM((BPW, D), jnp.float32),
            pltpu.SemaphoreType.DMA,
            pltpu.SemaphoreType.DMA,
        ],
        compiler_params=pltpu.CompilerParams(use_tc_tiling_on_sc=False),
    )


def _mlp_body(e_ref, w1_ref, b1_ref, w2_ref, b2_ref, out_ref):
    h = jnp.dot(e_ref[0], w1_ref[0:D], preferred_element_type=jnp.float32)
    h += jnp.dot(e_ref[1], w1_ref[D:2 * D], preferred_element_type=jnp.float32)
    h += jnp.dot(e_ref[2], w1_ref[2 * D:3 * D], preferred_element_type=jnp.float32)
    h += jnp.dot(e_ref[3], w1_ref[3 * D:4 * D], preferred_element_type=jnp.float32)
    h = jnp.maximum(h + b1_ref[...], 0.0)
    out_ref[...] = jnp.dot(h, w2_ref[...], preferred_element_type=jnp.float32) + b2_ref[...]


def _mlp(e, w1, b1, w2, b2, blk=2048):
    grid = (B // blk,)
    return pl.pallas_call(
        _mlp_body,
        grid=grid,
        in_specs=[
            pl.BlockSpec((4, blk, D), lambda i: (0, i, 0)),
            pl.BlockSpec((4 * D, H), lambda i: (0, 0)),
            pl.BlockSpec((1, H), lambda i: (0, 0)),
            pl.BlockSpec((H, D), lambda i: (0, 0)),
            pl.BlockSpec((1, D), lambda i: (0, 0)),
        ],
        out_specs=pl.BlockSpec((blk, D), lambda i: (i, 0)),
        out_shape=jax.ShapeDtypeStruct((B, D), jnp.float32),
    )(e, w1, b1, w2, b2)


def kernel(item_id, category, category2, category3,
           item_table, cat1_table, cat2_table, cat3_table,
           W1, b1, W2, b2):
    e = _sc_gather()(item_id, category, category2, category3,
                     item_table, cat1_table, cat2_table, cat3_table)
    return _mlp(e, W1, b1.reshape(1, H), W2, b2.reshape(1, D))
